# baseline (speedup 1.0000x reference)
import functools

import jax
import jax.numpy as jnp
from jax import lax
from jax.experimental import pallas as pl
from jax.experimental.pallas import tpu as pltpu
from jax.experimental.pallas import tpu_sc as plsc

B = 1024
D = 512
N = 100000
K = 32
G = 128
NP = 100352
NG = NP // G
NEG = -3.0e38
BIGI = 2**30

RT = 256
CB = 2048
NRB = B // RT
NCB = NP // CB
GPB = CB // G


NGP = 896


def _k1_body(x_ref, w_ref, s_ref, gm_ref):
    j = pl.program_id(0)
    i = pl.program_id(1)
    s = jnp.dot(x_ref[...], w_ref[...], preferred_element_type=jnp.float32)
    gcol = j * CB + lax.broadcasted_iota(jnp.int32, (RT, CB), 1)
    s = jnp.where(gcol < N, s, NEG)
    s_ref[...] = s
    gml = jnp.max(s.reshape(RT, GPB, G), axis=2)
    gm8 = jnp.tile(gml, (1, 8))
    roff = pl.multiple_of(i * RT, RT)
    loff = pl.multiple_of((j // 8) * 128, 128)
    slot = lax.broadcasted_iota(jnp.int32, (RT, 128), 1) // GPB
    cur = gm_ref[pl.ds(roff, RT), pl.ds(loff, 128)]
    gm_ref[pl.ds(roff, RT), pl.ds(loff, 128)] = jnp.where(
        slot == j % 8, gm8, cur)


def _scores_and_groupmax(x, wt):
    return pl.pallas_call(
        _k1_body,
        grid=(NCB, NRB),
        in_specs=[
            pl.BlockSpec((RT, D), lambda j, i: (i, 0)),
            pl.BlockSpec((D, CB), lambda j, i: (0, j)),
        ],
        out_specs=[
            pl.BlockSpec((RT, CB), lambda j, i: (i, j)),
            pl.BlockSpec((B, NGP), lambda j, i: (0, 0)),
        ],
        out_shape=[
            jax.ShapeDtypeStruct((B, NP), jnp.float32),
            jax.ShapeDtypeStruct((B, NGP), jnp.float32),
        ],
    )(x, wt)


def _k2_body(gm_ref, flat_ref, gid_ref):
    i = pl.program_id(0)
    lane = lax.broadcasted_iota(jnp.int32, (RT, NGP), 1)
    row = i * RT + lax.broadcasted_iota(jnp.int32, (RT, 1), 0)
    kio = lax.broadcasted_iota(jnp.int32, (RT, K), 1)
    g0 = jnp.where(lane < NG, gm_ref[...], NEG)

    def step(k, carry):
        g, gids = carry
        m = jnp.max(g, axis=1, keepdims=True)
        c = jnp.min(jnp.where(g == m, lane, BIGI), axis=1, keepdims=True)
        gids = jnp.where(kio == k, c, gids)
        return jnp.where(lane == c, NEG, g), gids

    _, gids = lax.fori_loop(0, K, step,
                            (g0, jnp.zeros((RT, K), jnp.int32)))
    gid_ref[...] = gids
    flat_ref[...] = row * NG + gids


def _top_groups(gm):
    return pl.pallas_call(
        _k2_body,
        grid=(NRB,),
        in_specs=[pl.BlockSpec((RT, NGP), lambda i: (i, 0))],
        out_specs=[
            pl.BlockSpec((RT, K), lambda i: (i, 0)),
            pl.BlockSpec((RT, K), lambda i: (i, 0)),
        ],
        out_shape=[
            jax.ShapeDtypeStruct((B, K), jnp.int32),
            jax.ShapeDtypeStruct((B, K), jnp.int32),
        ],
    )(gm)


NCAND = K * G


def _k4_body(cand_ref, gid_ref, vals_ref, idx_ref):
    gids = gid_ref[...]
    lane = lax.broadcasted_iota(jnp.int32, (RT, NCAND), 1)
    iglob = jnp.repeat(gids, G, axis=1) * G + lane % G
    kio = lax.broadcasted_iota(jnp.int32, (RT, K), 1)

    def step(k, carry):
        s, vals, idx = carry
        m = jnp.max(s, axis=1, keepdims=True)
        c = jnp.min(jnp.where(s == m, iglob, BIGI), axis=1, keepdims=True)
        vals = jnp.where(kio == k, m, vals)
        idx = jnp.where(kio == k, c, idx)
        return jnp.where(iglob == c, NEG, s), vals, idx

    _, vals, idx = lax.fori_loop(
        0, K, step,
        (cand_ref[...], jnp.zeros((RT, K), jnp.float32),
         jnp.zeros((RT, K), jnp.int32)))
    vals_ref[...] = vals
    idx_ref[...] = idx


def _top_candidates(cand, gids):
    return pl.pallas_call(
        _k4_body,
        grid=(NRB,),
        in_specs=[
            pl.BlockSpec((RT, NCAND), lambda i: (i, 0)),
            pl.BlockSpec((RT, K), lambda i: (i, 0)),
        ],
        out_specs=[
            pl.BlockSpec((RT, K), lambda i: (i, 0)),
            pl.BlockSpec((RT, K), lambda i: (i, 0)),
        ],
        out_shape=[
            jax.ShapeDtypeStruct((B, K), jnp.float32),
            jax.ShapeDtypeStruct((B, K), jnp.int32),
        ],
    )(cand, gids)


NW = 32


def _sc_gather(table, idx2d, rows_per_dma, n_dma, row_w):
    per_w = n_dma * rows_per_dma

    @functools.partial(
        pl.kernel,
        mesh=plsc.VectorSubcoreMesh(core_axis_name="c", subcore_axis_name="s"),
        out_type=jax.ShapeDtypeStruct((NW * per_w, row_w), jnp.float32),
        scratch_types=[
            pltpu.VMEM((n_dma, rows_per_dma), jnp.int32),
            pltpu.VMEM((rows_per_dma, row_w), jnp.float32),
            pltpu.VMEM((rows_per_dma, row_w), jnp.float32),
            pltpu.SemaphoreType.DMA,
            pltpu.SemaphoreType.DMA,
            pltpu.SemaphoreType.DMA,
            pltpu.SemaphoreType.DMA,
        ],
    )
    def k(tab_hbm, idx_hbm, out_hbm, idx_v, buf0, buf1, g0, g1, s0, s1):
        wid = lax.axis_index("s") * 2 + lax.axis_index("c")
        pltpu.sync_copy(idx_hbm.at[pl.ds(wid * n_dma, n_dma)], idx_v)
        bufs, gsems, ssems = (buf0, buf1), (g0, g1), (s0, s1)
        stores = [None, None]
        for c in range(n_dma):
            b = c % 2
            if stores[b] is not None:
                stores[b].wait()
            pltpu.async_copy(tab_hbm.at[idx_v.at[c]], bufs[b], gsems[b]).wait()
            stores[b] = pltpu.async_copy(
                bufs[b],
                out_hbm.at[pl.ds(wid * per_w + c * rows_per_dma, rows_per_dma)],
                ssems[b])
        for b in range(2):
            if stores[b] is not None:
                stores[b].wait()

    return k(table, idx2d)


def kernel(image_features, attr_features):
    wt = jnp.pad(attr_features, ((0, NP - N), (0, 0))).T
    scores, gm = _scores_and_groupmax(image_features, wt)
    flat_idx, gids = _top_groups(gm)
    seg_table = scores.reshape(B * NG, G)
    cand = _sc_gather(seg_table, flat_idx.reshape(NW * 8, 128), 128, 8, G)
    vals, idx = _top_candidates(cand.reshape(B, NCAND), gids)
    rows = _sc_gather(attr_features, idx.reshape(NW * 16, 64), 64, 16, D)
    return rows.reshape(B, K, D), vals


# speedup vs baseline: 1.2835x; 1.2835x over previous
import functools

import jax
import jax.numpy as jnp
from jax import lax
from jax.experimental import pallas as pl
from jax.experimental.pallas import tpu as pltpu
from jax.experimental.pallas import tpu_sc as plsc

B = 1024
D = 512
N = 100000
K = 32
G = 128
NP = 100352
NG = NP // G
NEG = -3.0e38
BIGI = 2**30

RT = 256
CB = 2048
NRB = B // RT
NCB = NP // CB
GPB = CB // G


def _k1_body(x_ref, w_ref, s_ref, gm_ref):
    j = pl.program_id(0)
    s = jnp.dot(x_ref[...], w_ref[...], preferred_element_type=jnp.float32)
    gcol = j * CB + lax.broadcasted_iota(jnp.int32, (RT, CB), 1)
    s = jnp.where(gcol < N, s, NEG)
    s3 = s.reshape(RT, GPB, G)
    s_ref[...] = s3
    gm_ref[...] = jnp.max(s3, axis=2)[None]


def _scores_and_groupmax(x, wt):
    return pl.pallas_call(
        _k1_body,
        grid=(NCB, NRB),
        in_specs=[
            pl.BlockSpec((RT, D), lambda j, i: (i, 0)),
            pl.BlockSpec((D, CB), lambda j, i: (0, j)),
        ],
        out_specs=[
            pl.BlockSpec((RT, GPB, G), lambda j, i: (i, j, 0)),
            pl.BlockSpec((1, RT, GPB), lambda j, i: (j, i, 0)),
        ],
        out_shape=[
            jax.ShapeDtypeStruct((B, NG, G), jnp.float32),
            jax.ShapeDtypeStruct((NCB, B, GPB), jnp.float32),
        ],
    )(x, wt)


def _k2_body(gm_ref, flat_ref, gid_ref):
    i = pl.program_id(0)
    lane = lax.broadcasted_iota(jnp.int32, (RT, NG), 1)
    row = i * RT + lax.broadcasted_iota(jnp.int32, (RT, 1), 0)
    kio = lax.broadcasted_iota(jnp.int32, (RT, K), 1)
    g0 = gm_ref[...]

    def step(k, carry):
        g, gids = carry
        m = jnp.max(g, axis=1, keepdims=True)
        c = jnp.min(jnp.where(g == m, lane, BIGI), axis=1, keepdims=True)
        gids = jnp.where(kio == k, c, gids)
        return jnp.where(lane == c, NEG, g), gids

    _, gids = lax.fori_loop(0, K, step,
                            (g0, jnp.zeros((RT, K), jnp.int32)))
    gid_ref[...] = gids
    flat_ref[...] = row * NG + gids


def _top_groups(gm):
    return pl.pallas_call(
        _k2_body,
        grid=(NRB,),
        in_specs=[pl.BlockSpec((RT, NG), lambda i: (i, 0))],
        out_specs=[
            pl.BlockSpec((RT, K), lambda i: (i, 0)),
            pl.BlockSpec((RT, K), lambda i: (i, 0)),
        ],
        out_shape=[
            jax.ShapeDtypeStruct((B, K), jnp.int32),
            jax.ShapeDtypeStruct((B, K), jnp.int32),
        ],
    )(gm)


NCAND = K * G


def _k4_body(cand_ref, gid_ref, vals_ref, idx_ref):
    gids = gid_ref[...]
    lane = lax.broadcasted_iota(jnp.int32, (RT, NCAND), 1)
    iglob = jnp.repeat(gids, G, axis=1) * G + lane % G
    kio = lax.broadcasted_iota(jnp.int32, (RT, K), 1)

    def step(k, carry):
        s, vals, idx = carry
        m = jnp.max(s, axis=1, keepdims=True)
        c = jnp.min(jnp.where(s == m, iglob, BIGI), axis=1, keepdims=True)
        vals = jnp.where(kio == k, m, vals)
        idx = jnp.where(kio == k, c, idx)
        return jnp.where(iglob == c, NEG, s), vals, idx

    _, vals, idx = lax.fori_loop(
        0, K, step,
        (cand_ref[...], jnp.zeros((RT, K), jnp.float32),
         jnp.zeros((RT, K), jnp.int32)))
    vals_ref[...] = vals
    idx_ref[...] = idx


def _top_candidates(cand, gids):
    return pl.pallas_call(
        _k4_body,
        grid=(NRB,),
        in_specs=[
            pl.BlockSpec((RT, NCAND), lambda i: (i, 0)),
            pl.BlockSpec((RT, K), lambda i: (i, 0)),
        ],
        out_specs=[
            pl.BlockSpec((RT, K), lambda i: (i, 0)),
            pl.BlockSpec((RT, K), lambda i: (i, 0)),
        ],
        out_shape=[
            jax.ShapeDtypeStruct((B, K), jnp.float32),
            jax.ShapeDtypeStruct((B, K), jnp.int32),
        ],
    )(cand, gids)


NW = 32


def _sc_gather(table, idx2d, rows_per_dma, n_dma, row_w):
    per_w = n_dma * rows_per_dma

    @functools.partial(
        pl.kernel,
        mesh=plsc.VectorSubcoreMesh(core_axis_name="c", subcore_axis_name="s"),
        out_type=jax.ShapeDtypeStruct((NW * per_w, row_w), jnp.float32),
        scratch_types=[
            pltpu.VMEM((n_dma, rows_per_dma), jnp.int32),
            pltpu.VMEM((rows_per_dma, row_w), jnp.float32),
            pltpu.VMEM((rows_per_dma, row_w), jnp.float32),
            pltpu.SemaphoreType.DMA,
            pltpu.SemaphoreType.DMA,
            pltpu.SemaphoreType.DMA,
            pltpu.SemaphoreType.DMA,
        ],
    )
    def k(tab_hbm, idx_hbm, out_hbm, idx_v, buf0, buf1, g0, g1, s0, s1):
        wid = lax.axis_index("s") * 2 + lax.axis_index("c")
        pltpu.sync_copy(idx_hbm.at[pl.ds(wid * n_dma, n_dma)], idx_v)
        bufs, gsems, ssems = (buf0, buf1), (g0, g1), (s0, s1)
        stores = [None, None]
        for c in range(n_dma):
            b = c % 2
            if stores[b] is not None:
                stores[b].wait()
            pltpu.async_copy(tab_hbm.at[idx_v.at[c]], bufs[b], gsems[b]).wait()
            stores[b] = pltpu.async_copy(
                bufs[b],
                out_hbm.at[pl.ds(wid * per_w + c * rows_per_dma, rows_per_dma)],
                ssems[b])
        for b in range(2):
            if stores[b] is not None:
                stores[b].wait()

    return k(table, idx2d)


def kernel(image_features, attr_features):
    wt = jnp.pad(attr_features, ((0, NP - N), (0, 0))).T
    scores, gm3 = _scores_and_groupmax(image_features, wt)
    gm = gm3.transpose(1, 0, 2).reshape(B, NG)
    flat_idx, gids = _top_groups(gm)
    seg_table = scores.reshape(B * NG, G)
    cand = _sc_gather(seg_table, flat_idx.reshape(NW * 8, 128), 128, 8, G)
    vals, idx = _top_candidates(cand.reshape(B, NCAND), gids)
    rows = _sc_gather(attr_features, idx.reshape(NW * 16, 64), 64, 16, D)
    return rows.reshape(B, K, D), vals
